# Initial kernel scaffold; baseline (speedup 1.0000x reference)
import jax
import jax.numpy as jnp
from jax.experimental import pallas as pl


def kernel(locs, data, neighbors, weight, bias):
    raise NotImplementedError("write your pallas kernel here")


# speedup vs baseline: 2.7890x; 2.7890x over previous
import functools

import jax
import jax.numpy as jnp
import numpy as np
from jax import lax
from jax.experimental import pallas as pl
from jax.experimental.pallas import tpu as pltpu
from jax.experimental.pallas import tpu_sc as plsc

_RADIUS = 1.0
_NORM = 15.0 / (np.pi * _RADIUS ** 6)

_NC, _NS, _L = 2, 16, 16
_NW = _NC * _NS
_G = 4


def _mm_body(d_ref, w_ref, o_ref):
    o_ref[...] = jnp.dot(d_ref[...], w_ref[...],
                         preferred_element_type=jnp.float32)


def _rsqrt(x):
    i = plsc.bitcast(x, jnp.int32)
    i = jnp.int32(0x5F3759DF) - lax.shift_right_logical(i, 1)
    r = plsc.bitcast(i, jnp.float32)
    for _ in range(3):
        r = r * (jnp.float32(1.5) - jnp.float32(0.5) * x * r * r)
    return r


def _make_sc_kernel(n_pad, m, c, P):
    NG = P // _G
    GM = _G * m
    mesh = plsc.VectorSubcoreMesh(core_axis_name="c", subcore_axis_name="s",
                                  num_cores=_NC, num_subcores=_NS)
    CS = c // _L

    @functools.partial(
        pl.kernel,
        out_type=jax.ShapeDtypeStruct((n_pad, c), jnp.float32),
        mesh=mesh,
        scratch_types=[
            pltpu.VMEM((n_pad,), jnp.float32),
            pltpu.VMEM((n_pad,), jnp.float32),
            pltpu.VMEM((n_pad,), jnp.float32),
            pltpu.VMEM((P * m,), jnp.int32),
            pltpu.VMEM((P * m,), jnp.int32),
            pltpu.VMEM((4, GM, c), jnp.float32),
            pltpu.VMEM((m + _L,), jnp.float32),
            pltpu.VMEM((c,), jnp.float32),
            pltpu.VMEM((2, _G, c), jnp.float32),
            pltpu.SemaphoreType.DMA,
            pltpu.SemaphoreType.DMA,
            pltpu.SemaphoreType.DMA,
            pltpu.SemaphoreType.DMA,
            pltpu.SemaphoreType.DMA,
            pltpu.SemaphoreType.DMA,
        ],
        compiler_params=pltpu.CompilerParams(needs_layout_passes=False),
    )
    def sc_kernel(xyz_hbm, neigh_hbm, pdata_hbm, bias_hbm, out_hbm,
                  x_v, y_v, z_v, neigh_v, idxc_v, rows_v, wbuf_v, bias_v,
                  stage_v, sem0, sem1, sem2, sem3, osem0, osem1):
        wid = lax.axis_index("s") * _NC + lax.axis_index("c")
        base = wid * P
        sems = (sem0, sem1, sem2, sem3)
        osems = (osem0, osem1)

        pltpu.sync_copy(xyz_hbm.at[pl.ds(0, n_pad)], x_v)
        pltpu.sync_copy(xyz_hbm.at[pl.ds(n_pad, n_pad)], y_v)
        pltpu.sync_copy(xyz_hbm.at[pl.ds(2 * n_pad, n_pad)], z_v)
        pltpu.sync_copy(neigh_hbm.at[pl.ds(base * m, P * m)], neigh_v)
        pltpu.sync_copy(bias_hbm, bias_v)

        def clip_body(i, carry):
            v = neigh_v[pl.ds(i * _L, _L)]
            idxc_v[pl.ds(i * _L, _L)] = jnp.minimum(
                jnp.maximum(v, jnp.int32(0)), jnp.int32(n_pad - 1))
            return carry
        lax.fori_loop(0, (P * m) // _L, clip_body, 0, unroll=8)

        def issue(g, b):
            pltpu.async_copy(
                pdata_hbm.at[idxc_v.at[pl.ds(g * GM, GM)]],
                rows_v.at[b], sems[b])

        def wait(g, b):
            pltpu.make_async_copy(
                pdata_hbm.at[idxc_v.at[pl.ds(g * GM, GM)]],
                rows_v.at[b], sems[b]).wait()

        issue(0, 0)
        issue(1, 1)
        issue(2, 2)

        def do_group(g, b):
            wait(g, b)
            @pl.when(g + 3 < NG)
            def _():
                issue(g + 3, (b + 3) % 4)
            ob = b % 2
            @pl.when(g >= 2)
            def _():
                pltpu.make_async_copy(
                    stage_v.at[ob],
                    out_hbm.at[pl.ds(base + (g - 2) * _G, _G)],
                    osems[ob]).wait()
            for p in range(_G):
                q = base + g * _G + p
                off = (g * _G + p) * m
                qs = jnp.full((_L,), q, dtype=jnp.int32)
                xo = plsc.load_gather(x_v, [qs])
                yo = plsc.load_gather(y_v, [qs])
                zo = plsc.load_gather(z_v, [qs])
                for h in range(m // _L):
                    io = neigh_v[pl.ds(off + h * _L, _L)]
                    ic = idxc_v[pl.ds(off + h * _L, _L)]
                    dx = plsc.load_gather(x_v, [ic]) - xo
                    dy = plsc.load_gather(y_v, [ic]) - yo
                    dz = plsc.load_gather(z_v, [ic]) - zo
                    d2 = jnp.maximum(dx * dx + dy * dy + dz * dz,
                                     jnp.float32(1e-12))
                    dist = d2 * _rsqrt(d2)
                    t = jnp.maximum(jnp.float32(_RADIUS) - dist,
                                    jnp.float32(0.0))
                    w = jnp.float32(_NORM) * t * t * t
                    w = jnp.where(io >= 0, w, jnp.float32(0.0))
                    wbuf_v[pl.ds(h * _L, _L)] = w

                acc = list(bias_v[pl.ds(k * _L, _L)] for k in range(CS))

                def macc8(i, a):
                    woff = i * 8
                    wv = wbuf_v[pl.ds(woff, _L)]
                    row0 = p * m + woff
                    a = list(a)
                    for j in range(8):
                        ws = wv[j]
                        for k in range(CS):
                            a[k] = a[k] + ws * rows_v[b, row0 + j,
                                                      pl.ds(k * _L, _L)]
                    return tuple(a)
                acc = lax.fori_loop(0, m // 8, macc8, tuple(acc))
                for k in range(CS):
                    stage_v[ob, p, pl.ds(k * _L, _L)] = acc[k]

            pltpu.async_copy(stage_v.at[ob],
                             out_hbm.at[pl.ds(base + g * _G, _G)],
                             osems[ob])

        def outer(i, carry):
            for bb in range(4):
                do_group(4 * i + bb, bb)
            return carry
        lax.fori_loop(0, NG // 4, outer, 0)
        pltpu.make_async_copy(
            stage_v.at[0], out_hbm.at[pl.ds(base + (NG - 2) * _G, _G)],
            osems[0]).wait()
        pltpu.make_async_copy(
            stage_v.at[1], out_hbm.at[pl.ds(base + (NG - 1) * _G, _G)],
            osems[1]).wait()

    return sc_kernel


def kernel(locs, data, neighbors, weight, bias):
    b, n, d = locs.shape
    m = neighbors.shape[2]
    c = data.shape[2]
    o = weight.shape[0]

    P = -(-n // (_NW * 4 * _G)) * (4 * _G)
    n_pad = _NW * P
    pad = n_pad - n

    data_p = jnp.pad(data[0], ((0, pad), (0, 0)))
    wt = weight[:, :, 0].T
    blk = 512
    pdata = pl.pallas_call(
        _mm_body,
        grid=(n_pad // blk,),
        in_specs=[pl.BlockSpec((blk, c), lambda i: (i, 0)),
                  pl.BlockSpec((c, o), lambda i: (0, 0))],
        out_specs=pl.BlockSpec((blk, o), lambda i: (i, 0)),
        out_shape=jax.ShapeDtypeStruct((n_pad, o), jnp.float32),
    )(data_p, wt)

    xyz = jnp.pad(locs[0].T, ((0, 0), (0, pad))).reshape(3 * n_pad)
    neigh_flat = jnp.pad(neighbors[0].reshape(n * m), (0, pad * m))

    sc = _make_sc_kernel(n_pad, m, o, P)
    out_pad = sc(xyz, neigh_flat, pdata, bias)
    return out_pad[:n].reshape(1, n, o)


# speedup vs baseline: 3.0302x; 1.0865x over previous
import functools

import jax
import jax.numpy as jnp
import numpy as np
from jax import lax
from jax.experimental import pallas as pl
from jax.experimental.pallas import tpu as pltpu
from jax.experimental.pallas import tpu_sc as plsc

_RADIUS = 1.0
_NORM = 15.0 / (np.pi * _RADIUS ** 6)

_NC, _NS, _L = 2, 16, 16
_NW = _NC * _NS
_G = 4
_F0 = 0.775


def _mm_body(d_ref, w_ref, o_ref):
    o_ref[...] = jnp.dot(d_ref[...], w_ref[...],
                         preferred_element_type=jnp.float32)


def _rsqrt(x):
    i = plsc.bitcast(x, jnp.int32)
    i = jnp.int32(0x5F3759DF) - lax.shift_right_logical(i, 1)
    r = plsc.bitcast(i, jnp.float32)
    for _ in range(3):
        r = r * (jnp.float32(1.5) - jnp.float32(0.5) * x * r * r)
    return r


def _make_sc_kernel(n_pad, m, c, P0, P1):
    NG0, NG1 = P0 // _G, P1 // _G
    GM = _G * m
    mesh = plsc.VectorSubcoreMesh(core_axis_name="c", subcore_axis_name="s",
                                  num_cores=_NC, num_subcores=_NS)
    CS = c // _L

    @functools.partial(
        pl.kernel,
        out_type=jax.ShapeDtypeStruct((n_pad, c), jnp.float32),
        mesh=mesh,
        scratch_types=[
            pltpu.VMEM((n_pad,), jnp.float32),
            pltpu.VMEM((n_pad,), jnp.float32),
            pltpu.VMEM((n_pad,), jnp.float32),
            pltpu.VMEM((P0 * m,), jnp.int32),
            pltpu.VMEM((P0 * m,), jnp.int32),
            pltpu.VMEM((2, GM, c), jnp.float32),
            pltpu.VMEM((m + _L,), jnp.float32),
            pltpu.VMEM((c,), jnp.float32),
            pltpu.VMEM((2, _G, c), jnp.float32),
            pltpu.SemaphoreType.DMA,
            pltpu.SemaphoreType.DMA,
            pltpu.SemaphoreType.DMA,
            pltpu.SemaphoreType.DMA,
        ],
        compiler_params=pltpu.CompilerParams(needs_layout_passes=False),
    )
    def sc_kernel(xyz_hbm, neigh_hbm, pdata_hbm, bias_hbm, out_hbm,
                  x_v, y_v, z_v, neigh_v, idxc_v, rows_v, wbuf_v, bias_v,
                  stage_v, sem0, sem1, osem0, osem1):
        ci = lax.axis_index("c")
        si = lax.axis_index("s")
        base = jnp.where(ci == 0, si * P0, _NS * P0 + si * P1)
        NGw = jnp.where(ci == 0, NG0, NG1)
        sems = (sem0, sem1)
        osems = (osem0, osem1)

        pltpu.sync_copy(xyz_hbm.at[pl.ds(0, n_pad)], x_v)
        pltpu.sync_copy(xyz_hbm.at[pl.ds(n_pad, n_pad)], y_v)
        pltpu.sync_copy(xyz_hbm.at[pl.ds(2 * n_pad, n_pad)], z_v)
        pltpu.sync_copy(neigh_hbm.at[pl.ds(base * m, P0 * m)], neigh_v)
        pltpu.sync_copy(bias_hbm, bias_v)

        def clip_body(i, carry):
            v = neigh_v[pl.ds(i * _L, _L)]
            idxc_v[pl.ds(i * _L, _L)] = jnp.minimum(
                jnp.maximum(v, jnp.int32(0)), jnp.int32(n_pad - 1))
            return carry
        lax.fori_loop(0, (P0 * m) // _L, clip_body, 0, unroll=8)

        def issue(g, b):
            pltpu.async_copy(
                pdata_hbm.at[idxc_v.at[pl.ds(g * GM, GM)]],
                rows_v.at[b], sems[b])

        def wait(g, b):
            pltpu.make_async_copy(
                pdata_hbm.at[idxc_v.at[pl.ds(g * GM, GM)]],
                rows_v.at[b], sems[b]).wait()

        issue(0, 0)
        issue(1, 1)

        def do_group(g, b):
            wait(g, b)
            @pl.when(g >= 2)
            def _():
                pltpu.make_async_copy(
                    stage_v.at[b],
                    out_hbm.at[pl.ds(base + (g - 2) * _G, _G)],
                    osems[b]).wait()
            for p in range(_G):
                q = base + g * _G + p
                off = (g * _G + p) * m
                qs = jnp.full((_L,), q, dtype=jnp.int32)
                xo = plsc.load_gather(x_v, [qs])
                yo = plsc.load_gather(y_v, [qs])
                zo = plsc.load_gather(z_v, [qs])
                for h in range(m // _L):
                    io = neigh_v[pl.ds(off + h * _L, _L)]
                    ic = idxc_v[pl.ds(off + h * _L, _L)]
                    dx = plsc.load_gather(x_v, [ic]) - xo
                    dy = plsc.load_gather(y_v, [ic]) - yo
                    dz = plsc.load_gather(z_v, [ic]) - zo
                    d2 = jnp.maximum(dx * dx + dy * dy + dz * dz,
                                     jnp.float32(1e-12))
                    dist = d2 * _rsqrt(d2)
                    t = jnp.maximum(jnp.float32(_RADIUS) - dist,
                                    jnp.float32(0.0))
                    w = jnp.float32(_NORM) * t * t * t
                    w = jnp.where(io >= 0, w, jnp.float32(0.0))
                    wbuf_v[pl.ds(h * _L, _L)] = w

                acc = list(bias_v[pl.ds(k * _L, _L)] for k in range(CS))

                def macc8(i, a):
                    woff = i * 8
                    wv = wbuf_v[pl.ds(woff, _L)]
                    row0 = p * m + woff
                    a = list(a)
                    for j in range(8):
                        ws = wv[j]
                        for k in range(CS):
                            a[k] = a[k] + ws * rows_v[b, row0 + j,
                                                      pl.ds(k * _L, _L)]
                    return tuple(a)
                acc = lax.fori_loop(0, m // 8, macc8, tuple(acc))
                for k in range(CS):
                    stage_v[b, p, pl.ds(k * _L, _L)] = acc[k]

            pltpu.async_copy(stage_v.at[b],
                             out_hbm.at[pl.ds(base + g * _G, _G)],
                             osems[b])

            @pl.when(g + 2 < NGw)
            def _():
                issue(g + 2, b)

        def outer(i, carry):
            do_group(2 * i, 0)
            do_group(2 * i + 1, 1)
            return carry
        lax.fori_loop(0, NGw // 2, outer, 0)
        pltpu.make_async_copy(
            stage_v.at[0], out_hbm.at[pl.ds(base + (NGw - 2) * _G, _G)],
            osems[0]).wait()
        pltpu.make_async_copy(
            stage_v.at[1], out_hbm.at[pl.ds(base + (NGw - 1) * _G, _G)],
            osems[1]).wait()

    return sc_kernel


def kernel(locs, data, neighbors, weight, bias):
    b, n, d = locs.shape
    m = neighbors.shape[2]
    c = data.shape[2]
    o = weight.shape[0]

    Pavg = -(-n // (_NW * 2 * _G)) * (2 * _G)
    P0 = int(round(_F0 * 2 * Pavg / (2 * _G))) * (2 * _G)
    P1 = 2 * Pavg - P0
    n_pad = _NS * (P0 + P1)
    pad = n_pad - n

    data_p = jnp.pad(data[0], ((0, pad), (0, 0)))
    wt = weight[:, :, 0].T
    blk = 512
    pdata = pl.pallas_call(
        _mm_body,
        grid=(n_pad // blk,),
        in_specs=[pl.BlockSpec((blk, c), lambda i: (i, 0)),
                  pl.BlockSpec((c, o), lambda i: (0, 0))],
        out_specs=pl.BlockSpec((blk, o), lambda i: (i, 0)),
        out_shape=jax.ShapeDtypeStruct((n_pad, o), jnp.float32),
    )(data_p, wt)

    xyz = jnp.pad(locs[0].T, ((0, 0), (0, pad))).reshape(3 * n_pad)
    neigh_flat = jnp.pad(neighbors[0].reshape(n * m),
                         (0, (pad + P0 - P1) * m))

    sc = _make_sc_kernel(n_pad, m, o, P0, P1)
    out_pad = sc(xyz, neigh_flat, pdata, bias)
    return out_pad[:n].reshape(1, n, o)
